# baseline (speedup 1.0000x reference)
import functools

import jax
import jax.numpy as jnp
from jax import lax
from jax.experimental import pallas as pl
from jax.experimental.pallas import tpu as pltpu
from jax.experimental.pallas import tpu_sc as plsc

B, S, D = 4, 8192, 2048


def kernel(h, inputs):
    mesh = plsc.VectorSubcoreMesh(core_axis_name="c", subcore_axis_name="s")

    @functools.partial(
        pl.kernel,
        out_type=jax.ShapeDtypeStruct((B, D), jnp.float32),
        name="probe",
        mesh=mesh,
        compiler_params=pltpu.CompilerParams(needs_layout_passes=False),
        scratch_types=[],
    )
    def k(h_hbm, tok_hbm, out_hbm):
        pass

    return k(h.reshape(B * S * 16, D // 16), inputs)


# speedup vs baseline: 10.8232x; 10.8232x over previous
import functools

import jax
import jax.numpy as jnp
from jax import lax
from jax.experimental import pallas as pl
from jax.experimental.pallas import tpu as pltpu
from jax.experimental.pallas import tpu_sc as plsc

B, S, D = 4, 8192, 2048
L = 16


def _pooler(h_rows, tokens):
    mesh = plsc.VectorSubcoreMesh(core_axis_name="c", subcore_axis_name="s")

    @functools.partial(
        pl.kernel,
        out_type=jax.ShapeDtypeStruct((B, D), jnp.float32),
        mesh=mesh,
        compiler_params=pltpu.CompilerParams(needs_layout_passes=False),
        scratch_types=[
            pltpu.VMEM((S,), jnp.int32),
            pltpu.VMEM((L, D), jnp.float32),
            pltpu.SemaphoreType.DMA,
        ],
    )
    def k(h_hbm, tok_hbm, out_hbm, row_v, gat_v, sem):
        cid = lax.axis_index("c")
        sid = lax.axis_index("s")
        wid = sid * 2 + cid

        @pl.when(wid < B)
        def _():
            b = wid
            pltpu.sync_copy(tok_hbm.at[b], row_v)

            def body(i, acc):
                x = row_v[pl.ds(i * L, L)]
                return acc + plsc.all_reduce_population_count(x != 0)

            cnt = lax.fori_loop(0, S // L, body, jnp.zeros((L,), jnp.int32))
            idx = cnt - 1
            idx = jnp.where(idx < 0, idx + S, idx)
            gidx = b * S + idx
            pltpu.async_copy(h_hbm.at[gidx], gat_v, sem).wait()
            pltpu.sync_copy(gat_v.at[pl.ds(0, 1)], out_hbm.at[pl.ds(b, 1)])

    return k(h_rows, tokens)


def kernel(h, inputs):
    return _pooler(h.reshape(B * S, D), inputs)


# speedup vs baseline: 11.5535x; 1.0675x over previous
import functools

import jax
import jax.numpy as jnp
from jax import lax
from jax.experimental import pallas as pl
from jax.experimental.pallas import tpu as pltpu
from jax.experimental.pallas import tpu_sc as plsc

B, S, D = 4, 8192, 2048
L = 16


def _pooler(h_rows, tokens):
    mesh = plsc.VectorSubcoreMesh(core_axis_name="c", subcore_axis_name="s")

    @functools.partial(
        pl.kernel,
        out_type=jax.ShapeDtypeStruct((B, D), jnp.float32),
        mesh=mesh,
        compiler_params=pltpu.CompilerParams(needs_layout_passes=False),
        scratch_types=[
            pltpu.VMEM((S,), jnp.int32),
            pltpu.VMEM((L, D), jnp.float32),
            pltpu.SemaphoreType.DMA,
        ],
    )
    def k(h_hbm, tok_hbm, out_hbm, row_v, gat_v, sem):
        cid = lax.axis_index("c")
        sid = lax.axis_index("s")
        wid = sid * 2 + cid

        @pl.when(wid < B)
        def _():
            b = wid
            pltpu.sync_copy(tok_hbm.at[b], row_v)

            U = 8

            def body(i, acc):
                base = i * (L * U)
                for u in range(U):
                    x = row_v[pl.ds(base + u * L, L)]
                    acc = acc + plsc.all_reduce_population_count(x != 0)
                return acc

            cnt = lax.fori_loop(0, S // (L * U), body,
                                jnp.zeros((L,), jnp.int32))
            idx = cnt - 1
            idx = jnp.where(idx < 0, idx + S, idx)
            gidx = b * S + idx
            pltpu.async_copy(h_hbm.at[gidx], gat_v, sem).wait()
            pltpu.sync_copy(gat_v.at[pl.ds(0, 1)], out_hbm.at[pl.ds(b, 1)])

    return k(h_rows, tokens)


def kernel(h, inputs):
    return _pooler(h.reshape(B * S, D), inputs)


# speedup vs baseline: 14.3495x; 1.2420x over previous
import functools

import jax
import jax.numpy as jnp
from jax import lax
from jax.experimental import pallas as pl
from jax.experimental.pallas import tpu as pltpu
from jax.experimental.pallas import tpu_sc as plsc

B, S, D = 4, 8192, 2048
L = 16


def _pooler(h_rows, tokens):
    mesh = plsc.VectorSubcoreMesh(core_axis_name="c", subcore_axis_name="s",
                                  num_cores=1)

    @functools.partial(
        pl.kernel,
        out_type=jax.ShapeDtypeStruct((B, D), jnp.float32),
        mesh=mesh,
        compiler_params=pltpu.CompilerParams(needs_layout_passes=False,
                                             skip_device_barrier=True),
        scratch_types=[
            pltpu.VMEM((S,), jnp.int32),
        ],
    )
    def k(h_hbm, tok_hbm, out_hbm, row_v):
        sid = lax.axis_index("s")

        @pl.when(sid < B)
        def _():
            b = sid
            pltpu.sync_copy(tok_hbm.at[b], row_v)

            U = 8

            def body(i, acc):
                base = i * (L * U)
                for u in range(U):
                    x = row_v[pl.ds(base + u * L, L)]
                    acc = acc + (x != 0).astype(jnp.int32)
                return acc

            lane_cnt = lax.fori_loop(0, S // (L * U), body,
                                     jnp.zeros((L,), jnp.int32))
            cnt = jnp.sum(lane_cnt)
            idx = cnt - 1
            idx = jnp.where(idx < 0, idx + S, idx)
            pltpu.sync_copy(h_hbm.at[b * S + idx], out_hbm.at[b])

    return k(h_rows, tokens)


def kernel(h, inputs):
    return _pooler(h.reshape(B * S, D), inputs)
